# baseline (speedup 1.0000x reference)
import jax
import jax.numpy as jnp
from jax.experimental import pallas as pl


def _final_matmul_kernel(h_ref, w_ref, b_ref, o_ref):
    o_ref[...] = h_ref[...] @ w_ref[...] + b_ref[0]


def _batchnorm(x, g, b, eps=1e-5):
    mu = jnp.mean(x, axis=0)
    var = jnp.var(x, axis=0)
    return (x - mu) * jax.lax.rsqrt(var + eps) * g + b


def _gcn_conv(x, src, dst, ew, W, b):
    n = x.shape[0]
    loop = jnp.arange(n, dtype=src.dtype)
    s = jnp.concatenate([src, loop])
    d = jnp.concatenate([dst, loop])
    w = jnp.concatenate([ew, jnp.ones((n,), x.dtype)])
    deg = jnp.zeros((n,), x.dtype).at[d].add(w)
    dinv = jnp.where(deg > 0, jax.lax.rsqrt(deg), 0.0)
    norm = dinv[s] * w * dinv[d]
    xw = x @ W
    msgs = jnp.take(xw, s, axis=0) * norm[:, None]
    out = jnp.zeros((n, W.shape[1]), x.dtype).at[d].add(msgs)
    return out + b


def kernel(x, edge_index, edge_attr, bn_g, bn_b, W1, b1, cbn1_g, cbn1_b,
           W2, b2, cbn2_g, cbn2_b, W3, b3, cbn3_g, cbn3_b,
           W4, b4, cbn4_g, cbn4_b, out_W, out_b):
    src = edge_index[0]
    dst = edge_index[1]
    h = _batchnorm(x[:, 0:4], bn_g, bn_b)
    h = _gcn_conv(h, src, dst, edge_attr, W1, b1)
    h = jax.nn.relu(_batchnorm(h, cbn1_g, cbn1_b))
    h = _gcn_conv(h, src, dst, edge_attr, W2, b2)
    h = jax.nn.relu(_batchnorm(h, cbn2_g, cbn2_b))
    h = _gcn_conv(h, src, dst, edge_attr, W3, b3)
    h = jax.nn.relu(_batchnorm(h, cbn3_g, cbn3_b))
    h = _gcn_conv(h, src, dst, edge_attr, W4, b4)
    h = jax.nn.relu(_batchnorm(h, cbn4_g, cbn4_b))

    n = h.shape[0]
    blk = 1000
    out = pl.pallas_call(
        _final_matmul_kernel,
        grid=(n // blk,),
        in_specs=[
            pl.BlockSpec((blk, 32), lambda i: (i, 0)),
            pl.BlockSpec((32, 1), lambda i: (0, 0)),
            pl.BlockSpec((1,), lambda i: (0,)),
        ],
        out_specs=pl.BlockSpec((blk, 1), lambda i: (i, 0)),
        out_shape=jax.ShapeDtypeStruct((n, 1), jnp.float32),
    )(h, out_W, out_b)
    return out.squeeze(-1)


# speedup vs baseline: 8.5640x; 8.5640x over previous
import jax
import jax.numpy as jnp
from jax import lax
from jax.experimental import pallas as pl
from jax.experimental.pallas import tpu as pltpu
from jax.experimental.pallas import tpu_sc as plsc

N = 100000
NPAD = 100352
E = 1600000
EPAD = 1605632
GROUPS = EPAD // 128
KG = 8
CH = KG * 128
RPT = NPAD // 16
BLK = 2048
GRID = NPAD // BLK
EPS = 1e-5
F32 = jnp.float32


def _sc_mesh():
    return plsc.VectorSubcoreMesh(core_axis_name="c", subcore_axis_name="s")


def _mo(v, m):
    return pl.multiple_of(v, m)


def _zero_span(src_v, dst_ref, row0):
    for k in range(6):
        pltpu.sync_copy(src_v.at[pl.ds(0, 1024)],
                        dst_ref.at[pl.ds(_mo(row0 + k * 1024, 128), 1024)])
    pltpu.sync_copy(src_v.at[pl.ds(0, 128)],
                    dst_ref.at[pl.ds(_mo(row0 + 6144, 128), 128)])


def _copy_span(acc, out_ref, row0, out_off):
    for k in range(6):
        pltpu.sync_copy(acc.at[pl.ds(_mo(row0 + k * 1024, 128), 1024)],
                        out_ref.at[pl.ds(_mo(out_off + row0 + k * 1024, 128), 1024)])
    pltpu.sync_copy(acc.at[pl.ds(_mo(row0 + 6144, 128), 128)],
                    out_ref.at[pl.ds(_mo(out_off + row0 + 6144, 128), 128)])


def _make_prop(num_slices, split):
    rounds = 1 if split else num_slices // 2
    out_rows = (2 if split else num_slices) * NPAD

    def body(y_hbm, src_hbm, dst_hbm, ew_hbm, out_hbm,
             src_v, dst_v, ew_v, rows_v, acc, gsem, ssem):
        cid = lax.axis_index("c")
        sid = lax.axis_index("s")
        row0 = sid * RPT
        for r in range(rounds):
            if split:
                out_off = cid * NPAD
            else:
                s = rounds * cid + r
                out_off = s * NPAD

            @plsc.parallel_loop(0, CH, step=1, unroll=8)
            def _z(i):
                rows_v[i, :] = jnp.zeros((16,), F32)

            _zero_span(rows_v, acc, row0)
            plsc.subcore_barrier()

            if split:
                wid = cid * 16 + sid
                n_chunks = GROUPS // 32 // KG
                ebase0 = wid * (GROUPS // 32) * 128
            else:
                n_chunks = GROUPS // 16 // KG
                ebase0 = sid * (GROUPS // 16) * 128

            if split:
                ytab = y_hbm
            else:
                ytab = y_hbm.at[pl.ds(_mo(out_off, NPAD), NPAD)]

            def chunk(ci, c):
                ebase = _mo(ebase0 + ci * CH, CH)
                pltpu.sync_copy(src_hbm.at[pl.ds(ebase, CH)], src_v)
                pltpu.sync_copy(ew_hbm.at[pl.ds(ebase, CH)], ew_v)
                pltpu.sync_copy(dst_hbm.at[pl.ds(_mo(ebase // 128, KG), KG)], dst_v)
                gd = [pltpu.async_copy(ytab.at[src_v.at[pl.ds(j * 128, 128)]],
                                       rows_v.at[pl.ds(j * 128, 128)], gsem)
                      for j in range(KG)]
                for d in gd:
                    d.wait()

                @plsc.parallel_loop(0, CH, step=16, unroll=2)
                def _m(i):
                    wv = ew_v[pl.ds(i, 16)]
                    for l in range(16):
                        rows_v[i + l, :] = rows_v[i + l, :] * wv[l]

                sd = [pltpu.async_copy(rows_v.at[pl.ds(j * 128, 128)],
                                       acc.at[dst_v.at[j]], ssem, add=True)
                      for j in range(KG)]
                for d in sd:
                    d.wait()
                return c

            lax.fori_loop(0, n_chunks, chunk, None)
            plsc.subcore_barrier()
            _copy_span(acc, out_hbm, row0, out_off)
            if r + 1 < rounds:
                plsc.subcore_barrier()

    return pl.kernel(
        body,
        out_type=jax.ShapeDtypeStruct((out_rows, 16), F32),
        mesh=_sc_mesh(),
        compiler_params=pltpu.CompilerParams(use_tc_tiling_on_sc=False),
        scratch_types=[
            pltpu.VMEM((CH,), jnp.int32),
            pltpu.VMEM((KG, 128), jnp.int32),
            pltpu.VMEM((CH,), F32),
            pltpu.VMEM((CH, 16), F32),
            pltpu.VMEM_SHARED((NPAD, 16), F32),
            pltpu.SemaphoreType.DMA,
            pltpu.SemaphoreType.DMA,
        ],
    )


def _rsqrt_exact(v):
    r = lax.rsqrt(v)
    return r * (1.5 - 0.5 * v * r * r)


def _rowmask(t):
    rid = pl.program_id(0) * BLK + lax.broadcasted_iota(jnp.int32, (BLK, 1), 0)
    return jnp.where(rid < N, t, 0.0)


def _accum_stats(st_ref, t):
    tm = _rowmask(t)

    @pl.when(pl.program_id(0) == 0)
    def _():
        st_ref[...] = jnp.zeros_like(st_ref)

    st_ref[...] += jnp.concatenate(
        [jnp.sum(tm, axis=0)[None, :], jnp.sum(tm * tm, axis=0)[None, :]], axis=0)


def _bn_affine(st_ref, g_ref, b_ref):
    mu = st_ref[0, :] * (1.0 / N)
    var = st_ref[1, :] * (1.0 / N) - mu * mu
    a = _rsqrt_exact(var + EPS) * g_ref[...]
    c = b_ref[...] - mu * a
    return a, c


def _a1(x_ref, st_ref):
    xb = x_ref[:, 0:4]

    @pl.when(pl.program_id(0) == 0)
    def _():
        st_ref[...] = jnp.zeros_like(st_ref)

    st_ref[...] += jnp.concatenate(
        [jnp.sum(xb, axis=0)[None, :], jnp.sum(xb * xb, axis=0)[None, :]], axis=0)


def _a2(x_ref, st_ref, g_ref, b_ref, deg_ref, w_ref, y_ref, dinv_ref):
    a, c = _bn_affine(st_ref, g_ref, b_ref)
    h4 = x_ref[:, 0:4] * a[None, :] + c[None, :]
    h = jnp.concatenate([h4, jnp.zeros((BLK, 12), F32)], axis=1)
    deg = deg_ref[0][:, 0] + deg_ref[1][:, 0] + 1.0
    dinv = jnp.where(deg > 0, _rsqrt_exact(deg), 0.0)
    xw = jnp.dot(h, w_ref[...], preferred_element_type=F32)
    y = xw * dinv[:, None]
    for s in range(4):
        y_ref[s] = y[:, s * 16:(s + 1) * 16]
    dinv_ref[...] = dinv[:, None]


def _make_gather_sum(ns):
    def body(s_ref, y_ref, dinv_ref, o_ref, st_ref):
        dv = dinv_ref[...]
        for s in range(ns):
            o_ref[:, s * 16:(s + 1) * 16] = (s_ref[s] + y_ref[s]) * dv
        _accum_stats(st_ref, o_ref[...])

    return body


def _make_apply_mm(ns_out):
    def body(o_ref, st_ref, g_ref, b_ref, w_ref, dinv_ref, y_ref):
        a, c = _bn_affine(st_ref, g_ref, b_ref)
        h = jnp.maximum(o_ref[...] * a[None, :] + c[None, :], 0.0)
        z = jnp.dot(h, w_ref[...], preferred_element_type=F32)
        y = z * dinv_ref[...]
        for s in range(ns_out):
            y_ref[s] = y[:, s * 16:(s + 1) * 16]

    return body


def _e2(o_ref, st_ref, g_ref, b_ref, w_ref, ob_ref, out_ref):
    a, c = _bn_affine(st_ref, g_ref, b_ref)
    h = jnp.maximum(o_ref[...] * a[None, :] + c[None, :], 0.0)
    out_ref[...] = jnp.dot(h, w_ref[...], preferred_element_type=F32) + ob_ref[0]


def _rows(d):
    return pl.BlockSpec((BLK, d), lambda i: (i, 0))


def _slices(ns):
    return pl.BlockSpec((ns, BLK, 16), lambda i: (0, i, 0))


def _whole(*shape):
    nd = len(shape)
    return pl.BlockSpec(shape, lambda i, _n=nd: (0,) * _n)


def _st(d):
    return pl.BlockSpec((2, d), lambda i: (0, 0))


def _gather_sum_call(S, y, dinv, ns):
    return pl.pallas_call(
        _make_gather_sum(ns), grid=(GRID,),
        in_specs=[_slices(ns), _slices(ns), _rows(1)],
        out_specs=[_rows(16 * ns), _st(16 * ns)],
        out_shape=[jax.ShapeDtypeStruct((NPAD, 16 * ns), F32),
                   jax.ShapeDtypeStruct((2, 16 * ns), F32)])(S, y, dinv)


def _apply_mm_call(o, st, g, b, W, dinv, ns_out):
    d_in = o.shape[1]
    return pl.pallas_call(
        _make_apply_mm(ns_out), grid=(GRID,),
        in_specs=[_rows(d_in), _st(d_in), _whole(d_in), _whole(d_in),
                  _whole(d_in, 16 * ns_out), _rows(1)],
        out_specs=_slices(ns_out),
        out_shape=jax.ShapeDtypeStruct((ns_out, NPAD, 16), F32))(
        o, st, g, b, W, dinv)


def kernel(x, edge_index, edge_attr, bn_g, bn_b, W1, b1, cbn1_g, cbn1_b,
           W2, b2, cbn2_g, cbn2_b, W3, b3, cbn3_g, cbn3_b,
           W4, b4, cbn4_g, cbn4_b, out_W, out_b):
    src = edge_index[0].astype(jnp.int32)
    dst = edge_index[1].astype(jnp.int32)
    epad = EPAD - E
    src_p = jnp.concatenate([src, jnp.zeros((epad,), jnp.int32)])
    dst2d = jnp.concatenate([dst, jnp.zeros((epad,), jnp.int32)]).reshape(GROUPS, 128)
    ew_p = jnp.concatenate([edge_attr, jnp.zeros((epad,), F32)])
    xp = jnp.pad(x, ((0, NPAD - N), (0, 0)))
    W1p = jnp.pad(W1, ((0, 12), (0, 0)))

    prop1 = _make_prop(1, split=True)
    prop8 = _make_prop(8, split=False)
    prop4 = _make_prop(4, split=False)
    prop2 = _make_prop(2, split=False)

    ones_tab = jnp.ones((NPAD, 16), F32)
    degS = prop1(ones_tab, src_p, dst2d, ew_p).reshape(2, NPAD, 16)

    st0 = pl.pallas_call(
        _a1, grid=(GRID,),
        in_specs=[_rows(22)], out_specs=_st(4),
        out_shape=jax.ShapeDtypeStruct((2, 4), F32))(xp)

    y1, dinv = pl.pallas_call(
        _a2, grid=(GRID,),
        in_specs=[_rows(22), _st(4), _whole(4), _whole(4), _slices(2),
                  _whole(16, 64)],
        out_specs=[_slices(4), _rows(1)],
        out_shape=[jax.ShapeDtypeStruct((4, NPAD, 16), F32),
                   jax.ShapeDtypeStruct((NPAD, 1), F32)])(
        xp, st0, bn_g, bn_b, degS, W1p)

    S1 = prop4(y1.reshape(4 * NPAD, 16), src_p, dst2d, ew_p).reshape(4, NPAD, 16)
    o1, st1 = _gather_sum_call(S1, y1, dinv, 4)
    y2 = _apply_mm_call(o1, st1, cbn1_g, cbn1_b, W2, dinv, 8)

    S2 = prop8(y2.reshape(8 * NPAD, 16), src_p, dst2d, ew_p).reshape(8, NPAD, 16)
    o2, st2 = _gather_sum_call(S2, y2, dinv, 8)
    y3 = _apply_mm_call(o2, st2, cbn2_g, cbn2_b, W3, dinv, 4)

    S3 = prop4(y3.reshape(4 * NPAD, 16), src_p, dst2d, ew_p).reshape(4, NPAD, 16)
    o3, st3 = _gather_sum_call(S3, y3, dinv, 4)
    y4 = _apply_mm_call(o3, st3, cbn3_g, cbn3_b, W4, dinv, 2)

    S4 = prop2(y4.reshape(2 * NPAD, 16), src_p, dst2d, ew_p).reshape(2, NPAD, 16)
    o4, st4 = _gather_sum_call(S4, y4, dinv, 2)

    res = pl.pallas_call(
        _e2, grid=(GRID,),
        in_specs=[_rows(32), _st(32), _whole(32), _whole(32),
                  _whole(32, 1), _whole(1)],
        out_specs=_rows(1),
        out_shape=jax.ShapeDtypeStruct((NPAD, 1), F32))(
        o4, st4, cbn4_g, cbn4_b, out_W, out_b)

    return res[:N, 0]
